# baseline (speedup 1.0000x reference)
import jax
import jax.numpy as jnp
from jax import lax
from jax.experimental import pallas as pl
from jax.experimental.pallas import tpu as pltpu

E = 8
TOP_K = 2
D = 1024
F = 2048
N = 2048

BLK = 128
NBMAX = 40
SLOT = NBMAX * BLK

_BF = jnp.bfloat16


def _router_body(lg_ref, i0_ref, i1_ref, w0_ref, w1_ref):
    logits = lg_ref[...]
    iota = lax.broadcasted_iota(jnp.int32, (N, E), 1)
    m0 = jnp.max(logits, axis=1)
    i0 = jnp.min(jnp.where(logits == m0[:, None], iota, E), axis=1)
    neg = jnp.float32(-3.0e38)
    masked = jnp.where(iota == i0[:, None], neg, logits)
    m1 = jnp.max(masked, axis=1)
    i1 = jnp.min(jnp.where(masked == m1[:, None], iota, E), axis=1)
    w1 = jax.nn.sigmoid(m1 - m0)
    w0 = 1.0 - w1
    i0_ref[...] = i0
    i1_ref[...] = i1
    w0_ref[...] = w0
    w1_ref[...] = w1


def _run_router(logits):
    return pl.pallas_call(
        _router_body,
        out_shape=(
            jax.ShapeDtypeStruct((N,), jnp.int32),
            jax.ShapeDtypeStruct((N,), jnp.int32),
            jax.ShapeDtypeStruct((N,), jnp.float32),
            jax.ShapeDtypeStruct((N,), jnp.float32),
        ),
    )(logits)


def _dispatch_metadata(i0, i1, w0, w1):
    e_flat = jnp.concatenate([i0, i1])
    tok = jnp.concatenate([jnp.arange(N, dtype=jnp.int32)] * 2)
    w_flat = jnp.concatenate([w0, w1])
    counts = jnp.sum(e_flat[:, None] == jnp.arange(E, dtype=jnp.int32)[None, :],
                     axis=0, dtype=jnp.int32)
    pcounts = ((counts + BLK - 1) // BLK) * BLK
    start = jnp.cumsum(counts) - counts
    pstart = jnp.cumsum(pcounts) - pcounts
    order = jnp.argsort(e_flat, stable=True)
    e_s = e_flat[order]
    tok_s = tok[order]
    w_s = w_flat[order]
    rank = jnp.arange(2 * N, dtype=jnp.int32) - start[e_s]
    pos = pstart[e_s] + rank
    sidx = jnp.zeros((SLOT,), jnp.int32).at[pos].set(tok_s)
    sw = jnp.zeros((SLOT,), jnp.float32).at[pos].set(w_s)
    total_padded = jnp.sum(pcounts)
    nb_used = total_padded // BLK
    block_ids = jnp.arange(NBMAX, dtype=jnp.int32)
    block_start = block_ids * BLK
    bvalid = (block_start < total_padded).astype(jnp.int32)
    bexp_raw = jnp.clip(
        jnp.searchsorted(pstart, block_start, side="right").astype(jnp.int32) - 1,
        0, E - 1)
    last_e = bexp_raw[nb_used - 1]
    bexp = jnp.where(bvalid == 1, bexp_raw, last_e)
    bsel = jnp.where(bvalid == 1, block_ids, nb_used - 1)
    return sidx.reshape(NBMAX, 1, BLK), sw.reshape(NBMAX, 1, BLK), bexp, bsel, bvalid


def _gather_body(x_r_ref, x_i_ref, sidx_ref, xs_r_ref, xs_i_ref):
    idx = sidx_ref[0, 0, :]
    iota = lax.broadcasted_iota(jnp.int32, (BLK, N), 1)
    oh = (idx[:, None] == iota).astype(jnp.float32).astype(_BF)
    xs_r_ref[...] = jnp.dot(oh, x_r_ref[...],
                            preferred_element_type=jnp.float32).astype(_BF)
    xs_i_ref[...] = jnp.dot(oh, x_i_ref[...],
                            preferred_element_type=jnp.float32).astype(_BF)


def _run_gather(x_r, x_i, sidx3):
    full = lambda b: (0, 0)
    return pl.pallas_call(
        _gather_body,
        grid=(NBMAX,),
        in_specs=[
            pl.BlockSpec((N, D), full),
            pl.BlockSpec((N, D), full),
            pl.BlockSpec((1, 1, BLK), lambda b: (b, 0, 0)),
        ],
        out_specs=[
            pl.BlockSpec((BLK, D), lambda b: (b, 0)),
            pl.BlockSpec((BLK, D), lambda b: (b, 0)),
        ],
        out_shape=(
            jax.ShapeDtypeStruct((SLOT, D), _BF),
            jax.ShapeDtypeStruct((SLOT, D), _BF),
        ),
        compiler_params=pltpu.CompilerParams(
            dimension_semantics=("arbitrary",),
        ),
    )(x_r, x_i, sidx3)


def _ffn_body(bexp_ref, bsel_ref, bvalid_ref,
              xs_r_ref, xs_i_ref, sw_ref,
              wgr_ref, wgi_ref, wvr_ref, wvi_ref, wdr_ref, wdi_ref,
              bgr_ref, bgi_ref, bvr_ref, bvi_ref, bdr_ref, bdi_ref,
              ys_r_ref, ys_i_ref):
    b = pl.program_id(0)

    @pl.when(bvalid_ref[b] == 1)
    def _compute():
        xr = xs_r_ref[...]
        xi = xs_i_ref[...]
        dn_nt = (((1,), (1,)), ((), ()))

        wgr = wgr_ref[0]
        wgi = wgi_ref[0]
        gr = (lax.dot_general(xr, wgr, dn_nt, preferred_element_type=jnp.float32)
              - lax.dot_general(xi, wgi, dn_nt, preferred_element_type=jnp.float32)
              + bgr_ref[0])
        gi = (lax.dot_general(xi, wgr, dn_nt, preferred_element_type=jnp.float32)
              + lax.dot_general(xr, wgi, dn_nt, preferred_element_type=jnp.float32)
              + bgi_ref[0])
        mag = jnp.sqrt(gr * gr + gi * gi + 1e-8)
        act = mag * jax.nn.sigmoid(mag)

        wvr = wvr_ref[0]
        wvi = wvi_ref[0]
        vr = (lax.dot_general(xr, wvr, dn_nt, preferred_element_type=jnp.float32)
              - lax.dot_general(xi, wvi, dn_nt, preferred_element_type=jnp.float32)
              + bvr_ref[0])
        vi = (lax.dot_general(xi, wvr, dn_nt, preferred_element_type=jnp.float32)
              + lax.dot_general(xr, wvi, dn_nt, preferred_element_type=jnp.float32)
              + bvi_ref[0])
        hr = (act * vr).astype(_BF)
        hi = (act * vi).astype(_BF)

        wdr = wdr_ref[0]
        wdi = wdi_ref[0]
        yr = (lax.dot_general(hr, wdr, dn_nt, preferred_element_type=jnp.float32)
              - lax.dot_general(hi, wdi, dn_nt, preferred_element_type=jnp.float32)
              + bdr_ref[0])
        yi = (lax.dot_general(hi, wdr, dn_nt, preferred_element_type=jnp.float32)
              + lax.dot_general(hr, wdi, dn_nt, preferred_element_type=jnp.float32)
              + bdi_ref[0])
        w = sw_ref[0, 0, :]
        ys_r_ref[...] = (w[:, None] * yr).astype(_BF)
        ys_i_ref[...] = (w[:, None] * yi).astype(_BF)


def _run_ffn(xs_r, xs_i, sw3, bexp, bsel, bvalid,
             Wg_r, Wg_i, bg_r, bg_i, Wv_r, Wv_i, bv_r, bv_i,
             Wd_r, Wd_i, bd_r, bd_i):
    xs_map = lambda b, be, bs, bv: (bs[b], 0)
    sw_map = lambda b, be, bs, bv: (bs[b], 0, 0)
    w_map = lambda b, be, bs, bv: (be[b], 0, 0)

    grid_spec = pltpu.PrefetchScalarGridSpec(
        num_scalar_prefetch=3,
        grid=(NBMAX,),
        in_specs=[
            pl.BlockSpec((BLK, D), xs_map),
            pl.BlockSpec((BLK, D), xs_map),
            pl.BlockSpec((1, 1, BLK), sw_map),
            pl.BlockSpec((1, F, D), w_map),
            pl.BlockSpec((1, F, D), w_map),
            pl.BlockSpec((1, F, D), w_map),
            pl.BlockSpec((1, F, D), w_map),
            pl.BlockSpec((1, D, F), w_map),
            pl.BlockSpec((1, D, F), w_map),
            pl.BlockSpec((1, 1, F), w_map),
            pl.BlockSpec((1, 1, F), w_map),
            pl.BlockSpec((1, 1, F), w_map),
            pl.BlockSpec((1, 1, F), w_map),
            pl.BlockSpec((1, 1, D), w_map),
            pl.BlockSpec((1, 1, D), w_map),
        ],
        out_specs=[
            pl.BlockSpec((BLK, D), xs_map),
            pl.BlockSpec((BLK, D), xs_map),
        ],
    )
    return pl.pallas_call(
        _ffn_body,
        grid_spec=grid_spec,
        out_shape=(
            jax.ShapeDtypeStruct((SLOT, D), _BF),
            jax.ShapeDtypeStruct((SLOT, D), _BF),
        ),
        compiler_params=pltpu.CompilerParams(
            dimension_semantics=("arbitrary",),
            vmem_limit_bytes=66048 * 1024,
        ),
    )(bexp, bsel, bvalid, xs_r, xs_i, sw3,
      Wg_r, Wg_i, Wv_r, Wv_i, Wd_r, Wd_i,
      bg_r.reshape(E, 1, F), bg_i.reshape(E, 1, F),
      bv_r.reshape(E, 1, F), bv_i.reshape(E, 1, F),
      bd_r.reshape(E, 1, D), bd_i.reshape(E, 1, D))


def _combine_body(bvalid_ref, sidx_ref, ys_r_ref, ys_i_ref,
                  out_r_ref, out_i_ref):
    b = pl.program_id(0)

    @pl.when(b == 0)
    def _init():
        out_r_ref[...] = jnp.zeros_like(out_r_ref)
        out_i_ref[...] = jnp.zeros_like(out_i_ref)

    @pl.when(bvalid_ref[b] == 1)
    def _acc():
        idx = sidx_ref[0, 0, :]
        iota = lax.broadcasted_iota(jnp.int32, (BLK, N), 1)
        oh = (idx[:, None] == iota).astype(jnp.float32).astype(_BF)
        dn_tn = (((0,), (0,)), ((), ()))
        out_r_ref[...] += lax.dot_general(
            oh, ys_r_ref[...], dn_tn,
            preferred_element_type=jnp.float32)
        out_i_ref[...] += lax.dot_general(
            oh, ys_i_ref[...], dn_tn,
            preferred_element_type=jnp.float32)


def _run_combine(sidx3, ys_r, ys_i, bvalid):
    grid_spec = pltpu.PrefetchScalarGridSpec(
        num_scalar_prefetch=1,
        grid=(NBMAX,),
        in_specs=[
            pl.BlockSpec((1, 1, BLK), lambda b, bv: (b, 0, 0)),
            pl.BlockSpec((BLK, D), lambda b, bv: (b, 0)),
            pl.BlockSpec((BLK, D), lambda b, bv: (b, 0)),
        ],
        out_specs=[
            pl.BlockSpec((N, D), lambda b, bv: (0, 0)),
            pl.BlockSpec((N, D), lambda b, bv: (0, 0)),
        ],
    )
    return pl.pallas_call(
        _combine_body,
        grid_spec=grid_spec,
        out_shape=(
            jax.ShapeDtypeStruct((N, D), jnp.float32),
            jax.ShapeDtypeStruct((N, D), jnp.float32),
        ),
        compiler_params=pltpu.CompilerParams(
            dimension_semantics=("arbitrary",),
        ),
    )(bvalid, sidx3, ys_r, ys_i)


def kernel(x_r, x_i, router_W, router_b, Wg_r, Wg_i, bg_r, bg_i,
           Wv_r, Wv_i, bv_r, bv_i, Wd_r, Wd_i, bd_r, bd_i):
    Bs, Hs, Ts, Ds = x_r.shape
    xr = x_r.reshape(N, D)
    xi = x_i.reshape(N, D)
    logits = jnp.concatenate([xr, xi], axis=-1) @ router_W.T + router_b
    i0, i1, w0, w1 = _run_router(logits)
    sidx3, sw3, bexp, bsel, bvalid = _dispatch_metadata(i0, i1, w0, w1)
    xs_r, xs_i = _run_gather(xr.astype(_BF), xi.astype(_BF), sidx3)
    ys_r, ys_i = _run_ffn(xs_r, xs_i, sw3, bexp, bsel, bvalid,
                          Wg_r.astype(_BF), Wg_i.astype(_BF), bg_r, bg_i,
                          Wv_r.astype(_BF), Wv_i.astype(_BF), bv_r, bv_i,
                          Wd_r.astype(_BF), Wd_i.astype(_BF), bd_r, bd_i)
    out_r, out_i = _run_combine(sidx3, ys_r, ys_i, bvalid)
    return (out_r.reshape(Bs, Hs, Ts, Ds), out_i.reshape(Bs, Hs, Ts, Ds))


# speedup vs baseline: 1.5828x; 1.5828x over previous
import jax
import jax.numpy as jnp
from jax import lax
from jax.experimental import pallas as pl
from jax.experimental.pallas import tpu as pltpu

E = 8
TOP_K = 2
D = 1024
F = 2048
N = 2048

BLK = 256
NBMAX = 24
SLOT = NBMAX * BLK
FH = 1024
NH = F // FH

_BF = jnp.bfloat16


def _router_body(lg_ref, i0_ref, i1_ref, w0_ref, w1_ref):
    logits = lg_ref[...]
    iota = lax.broadcasted_iota(jnp.int32, (N, E), 1)
    m0 = jnp.max(logits, axis=1)
    i0 = jnp.min(jnp.where(logits == m0[:, None], iota, E), axis=1)
    neg = jnp.float32(-3.0e38)
    masked = jnp.where(iota == i0[:, None], neg, logits)
    m1 = jnp.max(masked, axis=1)
    i1 = jnp.min(jnp.where(masked == m1[:, None], iota, E), axis=1)
    w1 = jax.nn.sigmoid(m1 - m0)
    w0 = 1.0 - w1
    i0_ref[...] = i0
    i1_ref[...] = i1
    w0_ref[...] = w0
    w1_ref[...] = w1


def _run_router(logits):
    return pl.pallas_call(
        _router_body,
        out_shape=(
            jax.ShapeDtypeStruct((N,), jnp.int32),
            jax.ShapeDtypeStruct((N,), jnp.int32),
            jax.ShapeDtypeStruct((N,), jnp.float32),
            jax.ShapeDtypeStruct((N,), jnp.float32),
        ),
    )(logits)


def _dispatch_metadata(i0, i1, w0, w1):
    e_flat = jnp.concatenate([i0, i1])
    tok = jnp.concatenate([jnp.arange(N, dtype=jnp.int32)] * 2)
    w_flat = jnp.concatenate([w0, w1])
    counts = jnp.sum(e_flat[:, None] == jnp.arange(E, dtype=jnp.int32)[None, :],
                     axis=0, dtype=jnp.int32)
    pcounts = ((counts + BLK - 1) // BLK) * BLK
    start = jnp.cumsum(counts) - counts
    pstart = jnp.cumsum(pcounts) - pcounts
    order = jnp.argsort(e_flat, stable=True)
    e_s = e_flat[order]
    tok_s = tok[order]
    w_s = w_flat[order]
    rank = jnp.arange(2 * N, dtype=jnp.int32) - start[e_s]
    pos = pstart[e_s] + rank
    sidx = jnp.zeros((SLOT,), jnp.int32).at[pos].set(tok_s)
    sw = jnp.zeros((SLOT,), jnp.float32).at[pos].set(w_s)
    total_padded = jnp.sum(pcounts)
    nb_used = total_padded // BLK
    block_ids = jnp.arange(NBMAX, dtype=jnp.int32)
    block_start = block_ids * BLK
    bvalid = (block_start < total_padded).astype(jnp.int32)
    bexp_raw = jnp.clip(
        jnp.searchsorted(pstart, block_start, side="right").astype(jnp.int32) - 1,
        0, E - 1)
    last_e = bexp_raw[nb_used - 1]
    bexp = jnp.where(bvalid == 1, bexp_raw, last_e)
    bsel = jnp.where(bvalid == 1, block_ids, nb_used - 1)
    return sidx.reshape(NBMAX, 1, BLK), sw.reshape(NBMAX, 1, BLK), bexp, bsel, bvalid


def _ffn_body(bexp_ref, bsel_ref, bvalid_ref,
              x_r_ref, x_i_ref, sidx_ref, sw_ref,
              wgr_ref, wgi_ref, wvr_ref, wvi_ref, wdr_ref, wdi_ref,
              bgr_ref, bgi_ref, bvr_ref, bvi_ref, bdr_ref, bdi_ref,
              out_r_ref, out_i_ref,
              oh_s, xsr_s, xsi_s, accr_s, acci_s):
    b = pl.program_id(0)
    h = pl.program_id(1)

    @pl.when((b == 0) & (h == 0))
    def _init():
        out_r_ref[...] = jnp.zeros_like(out_r_ref)
        out_i_ref[...] = jnp.zeros_like(out_i_ref)

    @pl.when(bvalid_ref[b] == 1)
    def _compute():
        @pl.when(h == 0)
        def _gather():
            idx = sidx_ref[0, 0, :]
            iota = lax.broadcasted_iota(jnp.int32, (BLK, N), 1)
            oh = (idx[:, None] == iota).astype(jnp.float32).astype(_BF)
            oh_s[...] = oh
            xsr_s[...] = jnp.dot(oh, x_r_ref[...],
                                 preferred_element_type=jnp.float32).astype(_BF)
            xsi_s[...] = jnp.dot(oh, x_i_ref[...],
                                 preferred_element_type=jnp.float32).astype(_BF)

        xr = xsr_s[...]
        xi = xsi_s[...]
        dn_nt = (((1,), (1,)), ((), ()))

        wgr = wgr_ref[0]
        wgi = wgi_ref[0]
        gr = (lax.dot_general(xr, wgr, dn_nt, preferred_element_type=jnp.float32)
              - lax.dot_general(xi, wgi, dn_nt, preferred_element_type=jnp.float32)
              + bgr_ref[0])
        gi = (lax.dot_general(xi, wgr, dn_nt, preferred_element_type=jnp.float32)
              + lax.dot_general(xr, wgi, dn_nt, preferred_element_type=jnp.float32)
              + bgi_ref[0])
        mag = jnp.sqrt(gr * gr + gi * gi + 1e-8)
        act = mag * jax.nn.sigmoid(mag)

        wvr = wvr_ref[0]
        wvi = wvi_ref[0]
        vr = (lax.dot_general(xr, wvr, dn_nt, preferred_element_type=jnp.float32)
              - lax.dot_general(xi, wvi, dn_nt, preferred_element_type=jnp.float32)
              + bvr_ref[0])
        vi = (lax.dot_general(xi, wvr, dn_nt, preferred_element_type=jnp.float32)
              + lax.dot_general(xr, wvi, dn_nt, preferred_element_type=jnp.float32)
              + bvi_ref[0])
        hr = (act * vr).astype(_BF)
        hi = (act * vi).astype(_BF)

        wdr = wdr_ref[0]
        wdi = wdi_ref[0]
        pyr = (lax.dot_general(hr, wdr, dn_nt, preferred_element_type=jnp.float32)
               - lax.dot_general(hi, wdi, dn_nt, preferred_element_type=jnp.float32))
        pyi = (lax.dot_general(hi, wdr, dn_nt, preferred_element_type=jnp.float32)
               + lax.dot_general(hr, wdi, dn_nt, preferred_element_type=jnp.float32))

        @pl.when(h == 0)
        def _acc0():
            accr_s[...] = pyr
            acci_s[...] = pyi

        @pl.when(h > 0)
        def _acc():
            accr_s[...] += pyr
            acci_s[...] += pyi

        @pl.when(h == NH - 1)
        def _scatter():
            w = sw_ref[0, 0, :]
            yr = accr_s[...] + bdr_ref[0]
            yi = acci_s[...] + bdi_ref[0]
            wyr = (w[:, None] * yr).astype(_BF)
            wyi = (w[:, None] * yi).astype(_BF)
            oh = oh_s[...]
            dn_tn = (((0,), (0,)), ((), ()))
            out_r_ref[...] += lax.dot_general(
                oh, wyr, dn_tn, preferred_element_type=jnp.float32)
            out_i_ref[...] += lax.dot_general(
                oh, wyi, dn_tn, preferred_element_type=jnp.float32)


def _run_ffn(x_r, x_i, sidx3, sw3, bexp, bsel, bvalid,
             Wg_r, Wg_i, bg_r, bg_i, Wv_r, Wv_i, bv_r, bv_i,
             Wd_r, Wd_i, bd_r, bd_i):
    def h_eff(b, h, bv):
        return jnp.where(bv[b] == 1, h, NH - 1)

    full = lambda b, h, *_: (0, 0)
    blk_map = lambda b, h, be, bs, bv: (bs[b], 0, 0)
    up_w = lambda b, h, be, bs, bv: (be[b], h_eff(b, h, bv), 0)
    dn_w = lambda b, h, be, bs, bv: (be[b], 0, h_eff(b, h, bv))
    up_b = lambda b, h, be, bs, bv: (be[b], 0, h_eff(b, h, bv))
    dn_b = lambda b, h, be, bs, bv: (be[b], 0, 0)

    grid_spec = pltpu.PrefetchScalarGridSpec(
        num_scalar_prefetch=3,
        grid=(NBMAX, NH),
        in_specs=[
            pl.BlockSpec((N, D), full),
            pl.BlockSpec((N, D), full),
            pl.BlockSpec((1, 1, BLK), blk_map),
            pl.BlockSpec((1, 1, BLK), blk_map),
            pl.BlockSpec((1, FH, D), up_w),
            pl.BlockSpec((1, FH, D), up_w),
            pl.BlockSpec((1, FH, D), up_w),
            pl.BlockSpec((1, FH, D), up_w),
            pl.BlockSpec((1, D, FH), dn_w),
            pl.BlockSpec((1, D, FH), dn_w),
            pl.BlockSpec((1, 1, FH), up_b),
            pl.BlockSpec((1, 1, FH), up_b),
            pl.BlockSpec((1, 1, FH), up_b),
            pl.BlockSpec((1, 1, FH), up_b),
            pl.BlockSpec((1, 1, D), dn_b),
            pl.BlockSpec((1, 1, D), dn_b),
        ],
        out_specs=[
            pl.BlockSpec((N, D), full),
            pl.BlockSpec((N, D), full),
        ],
        scratch_shapes=[
            pltpu.VMEM((BLK, N), _BF),
            pltpu.VMEM((BLK, D), _BF),
            pltpu.VMEM((BLK, D), _BF),
            pltpu.VMEM((BLK, D), jnp.float32),
            pltpu.VMEM((BLK, D), jnp.float32),
        ],
    )
    out_r, out_i = pl.pallas_call(
        _ffn_body,
        grid_spec=grid_spec,
        out_shape=(
            jax.ShapeDtypeStruct((N, D), jnp.float32),
            jax.ShapeDtypeStruct((N, D), jnp.float32),
        ),
        compiler_params=pltpu.CompilerParams(
            dimension_semantics=("arbitrary", "arbitrary"),
            vmem_limit_bytes=66048 * 1024,
        ),
    )(bexp, bsel, bvalid, x_r, x_i, sidx3, sw3,
      Wg_r, Wg_i, Wv_r, Wv_i, Wd_r, Wd_i,
      bg_r.reshape(E, 1, F), bg_i.reshape(E, 1, F),
      bv_r.reshape(E, 1, F), bv_i.reshape(E, 1, F),
      bd_r.reshape(E, 1, D), bd_i.reshape(E, 1, D))
    return out_r, out_i


def kernel(x_r, x_i, router_W, router_b, Wg_r, Wg_i, bg_r, bg_i,
           Wv_r, Wv_i, bv_r, bv_i, Wd_r, Wd_i, bd_r, bd_i):
    Bs, Hs, Ts, Ds = x_r.shape
    xr = x_r.reshape(N, D)
    xi = x_i.reshape(N, D)
    logits = jnp.concatenate([xr, xi], axis=-1) @ router_W.T + router_b
    i0, i1, w0, w1 = _run_router(logits)
    sidx3, sw3, bexp, bsel, bvalid = _dispatch_metadata(i0, i1, w0, w1)
    out_r, out_i = _run_ffn(xr.astype(_BF), xi.astype(_BF),
                            sidx3, sw3, bexp, bsel, bvalid,
                            Wg_r.astype(_BF), Wg_i.astype(_BF), bg_r, bg_i,
                            Wv_r.astype(_BF), Wv_i.astype(_BF), bv_r, bv_i,
                            Wd_r.astype(_BF), Wd_i.astype(_BF), bd_r, bd_i)
    return (out_r.reshape(Bs, Hs, Ts, Ds), out_i.reshape(Bs, Hs, Ts, Ds))
